# baseline (speedup 1.0000x reference)
import functools

import jax
import jax.numpy as jnp
from jax import lax
from jax.experimental import pallas as pl
from jax.experimental.pallas import tpu as pltpu
from jax.experimental.pallas import tpu_sc as plsc

BT = 256


def _mlp_body(sched_ref, x_ref, w1_hbm, w3_hbm, w2_hbm, y_ref, b1, b3, b2, s1, s3, s2):
    i = pl.program_id(0)
    first = sched_ref[0, i]
    nxt2_e = sched_ref[1, i]
    valid2 = sched_ref[2, i]
    sl = sched_ref[3, i]
    cur_e = sched_ref[4, i]
    e_g1 = sched_ref[5, i]
    valid_g1 = sched_ref[6, i]

    def issue(e, s):
        pltpu.make_async_copy(w1_hbm.at[e], b1.at[s], s1.at[s]).start()
        pltpu.make_async_copy(w3_hbm.at[e], b3.at[s], s3.at[s]).start()
        pltpu.make_async_copy(w2_hbm.at[e], b2.at[s], s2.at[s]).start()

    @pl.when(i == 0)
    def _():
        issue(cur_e, sl)

    @pl.when((i == 0) & (valid_g1 == 1))
    def _():
        issue(e_g1, 1)

    @pl.when((first == 1) & (valid2 == 1))
    def _():
        issue(nxt2_e, lax.rem(sl + 2, 3))

    @pl.when(first == 1)
    def _():
        pltpu.make_async_copy(w1_hbm.at[cur_e], b1.at[sl], s1.at[sl]).wait()
        pltpu.make_async_copy(w3_hbm.at[cur_e], b3.at[sl], s3.at[sl]).wait()
        pltpu.make_async_copy(w2_hbm.at[cur_e], b2.at[sl], s2.at[sl]).wait()

    x = x_ref[...].astype(jnp.bfloat16)
    h1 = jnp.dot(x, b1[sl].astype(jnp.bfloat16), preferred_element_type=jnp.float32)
    h3 = jnp.dot(x, b3[sl].astype(jnp.bfloat16), preferred_element_type=jnp.float32)
    h = (h1 * jax.nn.sigmoid(h1) * h3).astype(jnp.bfloat16)
    y_ref[...] = jnp.dot(
        h, b2[sl].astype(jnp.bfloat16), preferred_element_type=jnp.float32
    )


def _grouped_mlp(x_sorted, sched, w1, w3, w2, *, interpret=False):
    nrows, d = x_sorted.shape
    f = w1.shape[2]
    ntiles = nrows // BT
    grid_spec = pltpu.PrefetchScalarGridSpec(
        num_scalar_prefetch=1,
        grid=(ntiles,),
        in_specs=[
            pl.BlockSpec((BT, d), lambda i, s: (i, 0)),
            pl.BlockSpec(memory_space=pltpu.MemorySpace.HBM),
            pl.BlockSpec(memory_space=pltpu.MemorySpace.HBM),
            pl.BlockSpec(memory_space=pltpu.MemorySpace.HBM),
        ],
        out_specs=pl.BlockSpec((BT, d), lambda i, s: (i, 0)),
        scratch_shapes=[
            pltpu.VMEM((3, d, f), jnp.float32),
            pltpu.VMEM((3, d, f), jnp.float32),
            pltpu.VMEM((3, f, d), jnp.float32),
            pltpu.SemaphoreType.DMA((3,)),
            pltpu.SemaphoreType.DMA((3,)),
            pltpu.SemaphoreType.DMA((3,)),
        ],
    )
    return pl.pallas_call(
        _mlp_body,
        grid_spec=grid_spec,
        out_shape=jax.ShapeDtypeStruct((nrows, d), jnp.float32),
        interpret=interpret,
    )(sched, x_sorted, w1, w3, w2)


def _sc_dispatch_scatter(hidden_states, pos0, pos1, nrows):
    t, d = hidden_states.shape
    info = plsc.get_sparse_core_info()
    nw = info.num_cores * info.num_subcores
    assert t % nw == 0
    per_w = t // nw
    mesh = plsc.VectorSubcoreMesh(core_axis_name="c", subcore_axis_name="s")

    @functools.partial(
        pl.kernel,
        mesh=mesh,
        out_type=jax.ShapeDtypeStruct((nrows, d), jnp.float32),
        scratch_types=[
            pltpu.VMEM((per_w, d), jnp.float32),
            pltpu.VMEM((per_w,), jnp.int32),
            pltpu.VMEM((per_w,), jnp.int32),
            pltpu.SemaphoreType.DMA,
            pltpu.SemaphoreType.DMA,
        ],
    )
    def k(hs_hbm, p0_hbm, p1_hbm, out_hbm, xrows_v, i0_v, i1_v, sem0, sem1):
        wid = lax.axis_index("s") * info.num_cores + lax.axis_index("c")
        base = wid * per_w
        pltpu.sync_copy(hs_hbm.at[pl.ds(base, per_w)], xrows_v)
        pltpu.sync_copy(p0_hbm.at[pl.ds(base, per_w)], i0_v)
        pltpu.sync_copy(p1_hbm.at[pl.ds(base, per_w)], i1_v)
        c0 = pltpu.async_copy(xrows_v, out_hbm.at[i0_v], sem0)
        c1 = pltpu.async_copy(xrows_v, out_hbm.at[i1_v], sem1)
        c0.wait()
        c1.wait()

    return k(hidden_states, pos0, pos1)


def _sc_finalize_gather(yw, pos0, pos1, t, d):
    info = plsc.get_sparse_core_info()
    nw = info.num_cores * info.num_subcores
    assert t % nw == 0
    per_w = t // nw
    mesh = plsc.VectorSubcoreMesh(core_axis_name="c", subcore_axis_name="s")

    @functools.partial(
        pl.kernel,
        mesh=mesh,
        out_type=(
            jax.ShapeDtypeStruct((t, d), jnp.float32),
            jax.ShapeDtypeStruct((t, d), jnp.float32),
        ),
        scratch_types=[
            pltpu.VMEM((per_w,), jnp.int32),
            pltpu.VMEM((per_w,), jnp.int32),
            pltpu.VMEM((per_w, d), jnp.float32),
            pltpu.VMEM((per_w, d), jnp.float32),
            pltpu.SemaphoreType.DMA,
            pltpu.SemaphoreType.DMA,
        ],
    )
    def k(yw_hbm, p0_hbm, p1_hbm, g0_hbm, g1_hbm, i0_v, i1_v, a_v, b_v, sem0, sem1):
        wid = lax.axis_index("s") * info.num_cores + lax.axis_index("c")
        base = wid * per_w
        pltpu.sync_copy(p0_hbm.at[pl.ds(base, per_w)], i0_v)
        pltpu.sync_copy(p1_hbm.at[pl.ds(base, per_w)], i1_v)
        cp0 = pltpu.async_copy(yw_hbm.at[i0_v], a_v, sem0)
        cp1 = pltpu.async_copy(yw_hbm.at[i1_v], b_v, sem1)
        cp0.wait()
        cp1.wait()
        pltpu.sync_copy(a_v, g0_hbm.at[pl.ds(base, per_w)])
        pltpu.sync_copy(b_v, g1_hbm.at[pl.ds(base, per_w)])

    return k(yw, pos0, pos1)


def _add_body(a_ref, b_ref, wa_ref, wb_ref, o_ref):
    o_ref[...] = a_ref[...] * wa_ref[...] + b_ref[...] * wb_ref[...]


def _combine_add(g0, g1, w0, w1c):
    t, d = g0.shape
    bt = 256
    return pl.pallas_call(
        _add_body,
        grid=(t // bt,),
        in_specs=[
            pl.BlockSpec((bt, d), lambda i: (i, 0)),
            pl.BlockSpec((bt, d), lambda i: (i, 0)),
            pl.BlockSpec((bt, 1), lambda i: (i, 0)),
            pl.BlockSpec((bt, 1), lambda i: (i, 0)),
        ],
        out_specs=pl.BlockSpec((bt, d), lambda i: (i, 0)),
        out_shape=jax.ShapeDtypeStruct((t, d), jnp.float32),
    )(g0, g1, w0, w1c)


def _routing(topk_ids, topk_weights, t, e, k, ntiles, nrows):
    n = t * k
    e_flat = topk_ids.reshape(n)
    onehot = (e_flat[:, None] == jnp.arange(e, dtype=e_flat.dtype)[None, :]).astype(
        jnp.int32
    )
    ranks_inc = jnp.cumsum(onehot, axis=0)
    counts = ranks_inc[-1]
    padded = ((counts + BT - 1) // BT) * BT
    pad_start = jnp.concatenate(
        [jnp.zeros((1,), padded.dtype), jnp.cumsum(padded)[:-1]]
    )
    rank = jnp.sum(ranks_inc * onehot, axis=1) - 1
    base = jnp.sum(onehot * pad_start[None, :], axis=1)
    dest = (base + rank).astype(jnp.int32)
    tile_expert = (
        jnp.sum(
            (jnp.arange(ntiles, dtype=jnp.int32)[:, None] * BT
             >= pad_start[None, :].astype(jnp.int32)).astype(jnp.int32),
            axis=1,
        )
        - 1
    )
    tile_expert = jnp.clip(tile_expert, 0, e - 1)
    pos0 = dest[0::k]
    pos1 = dest[1::k]
    firsts = jnp.concatenate(
        [jnp.ones((1,), jnp.int32), (tile_expert[1:] != tile_expert[:-1]).astype(jnp.int32)]
    )
    grp = jnp.cumsum(firsts) - 1
    ngroups = grp[-1] + 1
    gid = jnp.arange(ntiles, dtype=jnp.int32)
    sel = (grp[None, :] == gid[:, None]).astype(jnp.int32) * firsts[None, :]
    ge = jnp.sum(sel * tile_expert[None, :], axis=1)
    nxt2_e = jnp.sum(
        ((grp[:, None] + 2) == gid[None, :]).astype(jnp.int32) * ge[None, :], axis=1
    )
    valid2 = (grp + 2 < ngroups).astype(jnp.int32)
    slot = grp % 3
    e_g1 = jnp.broadcast_to(ge[1], (ntiles,))
    valid_g1 = jnp.broadcast_to((ngroups > 1).astype(jnp.int32), (ntiles,))
    sched = jnp.stack(
        [firsts, nxt2_e, valid2, slot, tile_expert, e_g1, valid_g1]
    ).astype(jnp.int32)
    return pos0, pos1, sched


def kernel(hidden_states, topk_weights, topk_ids, w1, w3, w2):
    t, d = hidden_states.shape
    e = w1.shape[0]
    k = topk_ids.shape[1]
    n = t * k
    assert n % BT == 0 and k == 2
    ntiles = n // BT + e - 1
    nrows = ntiles * BT

    pos0, pos1, sched = _routing(topk_ids, topk_weights, t, e, k, ntiles, nrows)
    x_sorted = _sc_dispatch_scatter(hidden_states, pos0, pos1, nrows)
    yw = _grouped_mlp(x_sorted, sched, w1, w3, w2)
    g0, g1 = _sc_finalize_gather(yw, pos0, pos1, t, d)
    return _combine_add(g0, g1, topk_weights[:, 0:1], topk_weights[:, 1:2])
    g0, g1 = _sc_finalize_gather(yw, pos0, pos1, t, d)
    return _combine_add(g0, g1, topk_weights[:, 0:1], topk_weights[:, 1:2])


# speedup vs baseline: 1.0250x; 1.0250x over previous
import functools

import jax
import jax.numpy as jnp
from jax import lax
from jax.experimental import pallas as pl
from jax.experimental.pallas import tpu as pltpu
from jax.experimental.pallas import tpu_sc as plsc

BT = 256


def _mlp_body(sched_ref, x_ref, w1_hbm, w3_hbm, w2_hbm, y_ref, b1, b3, b2, s1, s3, s2):
    i = pl.program_id(0)
    first = sched_ref[0, i]
    nxt_e = sched_ref[1, i]
    valid = sched_ref[2, i]
    sl = sched_ref[3, i]
    cur_e = sched_ref[4, i]

    def issue(e, s):
        pltpu.make_async_copy(w1_hbm.at[e], b1.at[s], s1.at[s]).start()
        pltpu.make_async_copy(w3_hbm.at[e], b3.at[s], s3.at[s]).start()
        pltpu.make_async_copy(w2_hbm.at[e], b2.at[s], s2.at[s]).start()

    @pl.when(i == 0)
    def _():
        issue(cur_e, sl)

    @pl.when((first == 1) & (valid == 1))
    def _():
        issue(nxt_e, 1 - sl)

    @pl.when(first == 1)
    def _():
        pltpu.make_async_copy(w1_hbm.at[cur_e], b1.at[sl], s1.at[sl]).wait()
        pltpu.make_async_copy(w3_hbm.at[cur_e], b3.at[sl], s3.at[sl]).wait()
        pltpu.make_async_copy(w2_hbm.at[cur_e], b2.at[sl], s2.at[sl]).wait()

    x = x_ref[...].astype(jnp.bfloat16)
    h1 = jnp.dot(x, b1[sl].astype(jnp.bfloat16), preferred_element_type=jnp.float32)
    h3 = jnp.dot(x, b3[sl].astype(jnp.bfloat16), preferred_element_type=jnp.float32)
    h = (h1 * jax.nn.sigmoid(h1) * h3).astype(jnp.bfloat16)
    y_ref[...] = jnp.dot(
        h, b2[sl].astype(jnp.bfloat16), preferred_element_type=jnp.float32
    )


def _grouped_mlp(x_sorted, sched, w1, w3, w2, *, interpret=False):
    nrows, d = x_sorted.shape
    f = w1.shape[2]
    ntiles = nrows // BT
    grid_spec = pltpu.PrefetchScalarGridSpec(
        num_scalar_prefetch=1,
        grid=(ntiles,),
        in_specs=[
            pl.BlockSpec((BT, d), lambda i, s: (i, 0)),
            pl.BlockSpec(memory_space=pltpu.MemorySpace.HBM),
            pl.BlockSpec(memory_space=pltpu.MemorySpace.HBM),
            pl.BlockSpec(memory_space=pltpu.MemorySpace.HBM),
        ],
        out_specs=pl.BlockSpec((BT, d), lambda i, s: (i, 0)),
        scratch_shapes=[
            pltpu.VMEM((2, d, f), jnp.float32),
            pltpu.VMEM((2, d, f), jnp.float32),
            pltpu.VMEM((2, f, d), jnp.float32),
            pltpu.SemaphoreType.DMA((2,)),
            pltpu.SemaphoreType.DMA((2,)),
            pltpu.SemaphoreType.DMA((2,)),
        ],
    )
    return pl.pallas_call(
        _mlp_body,
        grid_spec=grid_spec,
        out_shape=jax.ShapeDtypeStruct((nrows, d), jnp.float32),
        interpret=interpret,
    )(sched, x_sorted, w1, w3, w2)


def _sc_dispatch_scatter(hidden_states, pos0, pos1, nrows):
    t, d = hidden_states.shape
    info = plsc.get_sparse_core_info()
    nw = info.num_cores * info.num_subcores
    assert t % nw == 0
    per_w = t // nw
    mesh = plsc.VectorSubcoreMesh(core_axis_name="c", subcore_axis_name="s")

    @functools.partial(
        pl.kernel,
        mesh=mesh,
        out_type=jax.ShapeDtypeStruct((nrows, d), jnp.float32),
        scratch_types=[
            pltpu.VMEM((per_w, d), jnp.float32),
            pltpu.VMEM((per_w,), jnp.int32),
            pltpu.VMEM((per_w,), jnp.int32),
            pltpu.SemaphoreType.DMA,
            pltpu.SemaphoreType.DMA,
        ],
    )
    def k(hs_hbm, p0_hbm, p1_hbm, out_hbm, xrows_v, i0_v, i1_v, sem0, sem1):
        wid = lax.axis_index("s") * info.num_cores + lax.axis_index("c")
        base = wid * per_w
        pltpu.sync_copy(hs_hbm.at[pl.ds(base, per_w)], xrows_v)
        pltpu.sync_copy(p0_hbm.at[pl.ds(base, per_w)], i0_v)
        pltpu.sync_copy(p1_hbm.at[pl.ds(base, per_w)], i1_v)
        c0 = pltpu.async_copy(xrows_v, out_hbm.at[i0_v], sem0)
        c1 = pltpu.async_copy(xrows_v, out_hbm.at[i1_v], sem1)
        c0.wait()
        c1.wait()

    return k(hidden_states, pos0, pos1)


def _sc_finalize_gather(yw, pos0, pos1, t, d):
    info = plsc.get_sparse_core_info()
    nw = info.num_cores * info.num_subcores
    assert t % nw == 0
    per_w = t // nw
    mesh = plsc.VectorSubcoreMesh(core_axis_name="c", subcore_axis_name="s")

    @functools.partial(
        pl.kernel,
        mesh=mesh,
        out_type=(
            jax.ShapeDtypeStruct((t, d), jnp.float32),
            jax.ShapeDtypeStruct((t, d), jnp.float32),
        ),
        scratch_types=[
            pltpu.VMEM((per_w,), jnp.int32),
            pltpu.VMEM((per_w,), jnp.int32),
            pltpu.VMEM((per_w, d), jnp.float32),
            pltpu.VMEM((per_w, d), jnp.float32),
            pltpu.SemaphoreType.DMA,
            pltpu.SemaphoreType.DMA,
        ],
    )
    def k(yw_hbm, p0_hbm, p1_hbm, g0_hbm, g1_hbm, i0_v, i1_v, a_v, b_v, sem0, sem1):
        wid = lax.axis_index("s") * info.num_cores + lax.axis_index("c")
        base = wid * per_w
        pltpu.sync_copy(p0_hbm.at[pl.ds(base, per_w)], i0_v)
        pltpu.sync_copy(p1_hbm.at[pl.ds(base, per_w)], i1_v)
        cp0 = pltpu.async_copy(yw_hbm.at[i0_v], a_v, sem0)
        cp1 = pltpu.async_copy(yw_hbm.at[i1_v], b_v, sem1)
        cp0.wait()
        cp1.wait()
        pltpu.sync_copy(a_v, g0_hbm.at[pl.ds(base, per_w)])
        pltpu.sync_copy(b_v, g1_hbm.at[pl.ds(base, per_w)])

    return k(yw, pos0, pos1)


def _add_body(a_ref, b_ref, wa_ref, wb_ref, o_ref):
    o_ref[...] = a_ref[...] * wa_ref[...] + b_ref[...] * wb_ref[...]


def _combine_add(g0, g1, w0, w1c):
    t, d = g0.shape
    bt = 256
    return pl.pallas_call(
        _add_body,
        grid=(t // bt,),
        in_specs=[
            pl.BlockSpec((bt, d), lambda i: (i, 0)),
            pl.BlockSpec((bt, d), lambda i: (i, 0)),
            pl.BlockSpec((bt, 1), lambda i: (i, 0)),
            pl.BlockSpec((bt, 1), lambda i: (i, 0)),
        ],
        out_specs=pl.BlockSpec((bt, d), lambda i: (i, 0)),
        out_shape=jax.ShapeDtypeStruct((t, d), jnp.float32),
    )(g0, g1, w0, w1c)


def _routing(topk_ids, topk_weights, t, e, k, ntiles, nrows):
    n = t * k
    e_flat = topk_ids.reshape(n)
    onehot = (e_flat[:, None] == jnp.arange(e, dtype=e_flat.dtype)[None, :]).astype(
        jnp.int32
    )
    ranks_inc = jnp.cumsum(onehot, axis=0)
    counts = ranks_inc[-1]
    padded = ((counts + BT - 1) // BT) * BT
    pad_start = jnp.concatenate(
        [jnp.zeros((1,), padded.dtype), jnp.cumsum(padded)[:-1]]
    )
    rank = jnp.sum(ranks_inc * onehot, axis=1) - 1
    base = jnp.sum(onehot * pad_start[None, :], axis=1)
    dest = (base + rank).astype(jnp.int32)
    tile_expert = (
        jnp.sum(
            (jnp.arange(ntiles, dtype=jnp.int32)[:, None] * BT
             >= pad_start[None, :].astype(jnp.int32)).astype(jnp.int32),
            axis=1,
        )
        - 1
    )
    tile_expert = jnp.clip(tile_expert, 0, e - 1)
    pos0 = dest[0::k]
    pos1 = dest[1::k]
    firsts = jnp.concatenate(
        [jnp.ones((1,), jnp.int32), (tile_expert[1:] != tile_expert[:-1]).astype(jnp.int32)]
    )
    grp = jnp.cumsum(firsts) - 1
    ngroups = grp[-1] + 1
    gid = jnp.arange(ntiles, dtype=jnp.int32)
    sel = (grp[None, :] == gid[:, None]).astype(jnp.int32) * firsts[None, :]
    ge = jnp.sum(sel * tile_expert[None, :], axis=1)
    nxt_e = jnp.sum(
        ((grp[:, None] + 1) == gid[None, :]).astype(jnp.int32) * ge[None, :], axis=1
    )
    valid = (grp + 1 < ngroups).astype(jnp.int32)
    slot = grp % 2
    sched = jnp.stack(
        [firsts, nxt_e, valid, slot, tile_expert]
    ).astype(jnp.int32)
    return pos0, pos1, sched


def kernel(hidden_states, topk_weights, topk_ids, w1, w3, w2):
    t, d = hidden_states.shape
    e = w1.shape[0]
    k = topk_ids.shape[1]
    n = t * k
    assert n % BT == 0 and k == 2
    ntiles = n // BT + e - 1
    nrows = ntiles * BT

    pos0, pos1, sched = _routing(topk_ids, topk_weights, t, e, k, ntiles, nrows)
    x_sorted = _sc_dispatch_scatter(hidden_states, pos0, pos1, nrows)
    yw = _grouped_mlp(x_sorted, sched, w1, w3, w2)
    g0, g1 = _sc_finalize_gather(yw, pos0, pos1, t, d)
    return _combine_add(g0, g1, topk_weights[:, 0:1], topk_weights[:, 1:2])
    g0, g1 = _sc_finalize_gather(yw, pos0, pos1, t, d)
    return _combine_add(g0, g1, topk_weights[:, 0:1], topk_weights[:, 1:2])


# speedup vs baseline: 1.0700x; 1.0439x over previous
import functools

import jax
import jax.numpy as jnp
from jax import lax
from jax.experimental import pallas as pl
from jax.experimental.pallas import tpu as pltpu
from jax.experimental.pallas import tpu_sc as plsc

BT = 256


def _mlp_body(sched_ref, x_ref, w1_hbm, w3_hbm, w2_hbm, y_ref, b1, b3, b2, s1, s3, s2):
    i = pl.program_id(0)
    first = sched_ref[0, i]
    nxt_e = sched_ref[1, i]
    valid = sched_ref[2, i]
    sl = sched_ref[3, i]
    cur_e = sched_ref[4, i]

    def issue(e, s):
        pltpu.make_async_copy(w1_hbm.at[e], b1.at[s], s1.at[s]).start()
        pltpu.make_async_copy(w3_hbm.at[e], b3.at[s], s3.at[s]).start()
        pltpu.make_async_copy(w2_hbm.at[e], b2.at[s], s2.at[s]).start()

    @pl.when(i == 0)
    def _():
        issue(cur_e, sl)

    @pl.when((first == 1) & (valid == 1))
    def _():
        issue(nxt_e, 1 - sl)

    @pl.when(first == 1)
    def _():
        pltpu.make_async_copy(w1_hbm.at[cur_e], b1.at[sl], s1.at[sl]).wait()
        pltpu.make_async_copy(w3_hbm.at[cur_e], b3.at[sl], s3.at[sl]).wait()
        pltpu.make_async_copy(w2_hbm.at[cur_e], b2.at[sl], s2.at[sl]).wait()

    x = x_ref[...].astype(jnp.bfloat16)
    h1 = jnp.dot(x, b1[sl].astype(jnp.bfloat16), preferred_element_type=jnp.float32)
    h3 = jnp.dot(x, b3[sl].astype(jnp.bfloat16), preferred_element_type=jnp.float32)
    h = (h1 * jax.nn.sigmoid(h1) * h3).astype(jnp.bfloat16)
    y_ref[...] = jnp.dot(
        h, b2[sl].astype(jnp.bfloat16), preferred_element_type=jnp.float32
    )


def _grouped_mlp(x_sorted, sched, w1, w3, w2, *, interpret=False):
    nrows, d = x_sorted.shape
    f = w1.shape[2]
    ntiles = nrows // BT
    grid_spec = pltpu.PrefetchScalarGridSpec(
        num_scalar_prefetch=1,
        grid=(ntiles,),
        in_specs=[
            pl.BlockSpec((BT, d), lambda i, s: (i, 0)),
            pl.BlockSpec(memory_space=pltpu.MemorySpace.HBM),
            pl.BlockSpec(memory_space=pltpu.MemorySpace.HBM),
            pl.BlockSpec(memory_space=pltpu.MemorySpace.HBM),
        ],
        out_specs=pl.BlockSpec((BT, d), lambda i, s: (i, 0)),
        scratch_shapes=[
            pltpu.VMEM((2, d, f), jnp.float32),
            pltpu.VMEM((2, d, f), jnp.float32),
            pltpu.VMEM((2, f, d), jnp.float32),
            pltpu.SemaphoreType.DMA((2,)),
            pltpu.SemaphoreType.DMA((2,)),
            pltpu.SemaphoreType.DMA((2,)),
        ],
    )
    return pl.pallas_call(
        _mlp_body,
        grid_spec=grid_spec,
        out_shape=jax.ShapeDtypeStruct((nrows, d), jnp.float32),
        interpret=interpret,
    )(sched, x_sorted, w1, w3, w2)


def _sc_dispatch_scatter(hidden_states, pos0, pos1, nrows):
    t, d = hidden_states.shape
    info = plsc.get_sparse_core_info()
    nw = info.num_cores * info.num_subcores
    assert t % nw == 0
    per_w = t // nw
    mesh = plsc.VectorSubcoreMesh(core_axis_name="c", subcore_axis_name="s")

    @functools.partial(
        pl.kernel,
        mesh=mesh,
        out_type=jax.ShapeDtypeStruct((nrows, d), jnp.float32),
        scratch_types=[
            pltpu.VMEM((per_w, d), jnp.float32),
            pltpu.VMEM((per_w,), jnp.int32),
            pltpu.VMEM((per_w,), jnp.int32),
            pltpu.SemaphoreType.DMA,
            pltpu.SemaphoreType.DMA,
        ],
    )
    def k(hs_hbm, p0_hbm, p1_hbm, out_hbm, xrows_v, i0_v, i1_v, sem0, sem1):
        wid = lax.axis_index("s") * info.num_cores + lax.axis_index("c")
        base = wid * per_w
        pltpu.sync_copy(hs_hbm.at[pl.ds(base, per_w)], xrows_v)
        pltpu.sync_copy(p0_hbm.at[pl.ds(base, per_w)], i0_v)
        pltpu.sync_copy(p1_hbm.at[pl.ds(base, per_w)], i1_v)
        c0 = pltpu.async_copy(xrows_v, out_hbm.at[i0_v], sem0)
        c1 = pltpu.async_copy(xrows_v, out_hbm.at[i1_v], sem1)
        c0.wait()
        c1.wait()

    return k(hidden_states, pos0, pos1)


def _sc_finalize_combine(yw, pos0, pos1, w0x, w1x, t, d):
    info = plsc.get_sparse_core_info()
    nw = info.num_cores * info.num_subcores
    assert t % nw == 0
    per_w = t // nw
    nlanes = d // 16
    mesh = plsc.VectorSubcoreMesh(core_axis_name="c", subcore_axis_name="s")

    @functools.partial(
        pl.kernel,
        mesh=mesh,
        out_type=jax.ShapeDtypeStruct((t, d), jnp.float32),
        scratch_types=[
            pltpu.VMEM((per_w,), jnp.int32),
            pltpu.VMEM((per_w,), jnp.int32),
            pltpu.VMEM((per_w, 16), jnp.float32),
            pltpu.VMEM((per_w, 16), jnp.float32),
            pltpu.VMEM((per_w, d), jnp.float32),
            pltpu.VMEM((per_w, d), jnp.float32),
            pltpu.SemaphoreType.DMA,
            pltpu.SemaphoreType.DMA,
        ],
    )
    def k(yw_hbm, p0_hbm, p1_hbm, w0_hbm, w1_hbm, out_hbm,
          i0_v, i1_v, w0_v, w1_v, a_v, b_v, sem0, sem1):
        wid = lax.axis_index("s") * info.num_cores + lax.axis_index("c")
        base = wid * per_w
        pltpu.sync_copy(p0_hbm.at[pl.ds(base, per_w)], i0_v)
        pltpu.sync_copy(p1_hbm.at[pl.ds(base, per_w)], i1_v)
        cp0 = pltpu.async_copy(yw_hbm.at[i0_v], a_v, sem0)
        cp1 = pltpu.async_copy(yw_hbm.at[i1_v], b_v, sem1)
        pltpu.sync_copy(w0_hbm.at[pl.ds(base, per_w)], w0_v)
        pltpu.sync_copy(w1_hbm.at[pl.ds(base, per_w)], w1_v)
        cp0.wait()
        cp1.wait()

        def row_fma(r, carry):
            wv0 = w0_v[r, :]
            wv1 = w1_v[r, :]
            for c in range(nlanes):
                s = pl.ds(c * 16, 16)
                a_v[r, s] = a_v[r, s] * wv0 + b_v[r, s] * wv1
            return carry

        lax.fori_loop(0, per_w, row_fma, 0)
        pltpu.sync_copy(a_v, out_hbm.at[pl.ds(base, per_w)])

    return k(yw, pos0, pos1, w0x, w1x)


def _routing(topk_ids, topk_weights, t, e, k, ntiles, nrows):
    n = t * k
    e_flat = topk_ids.reshape(n)
    onehot = (e_flat[:, None] == jnp.arange(e, dtype=e_flat.dtype)[None, :]).astype(
        jnp.int32
    )
    ranks_inc = jnp.cumsum(onehot, axis=0)
    counts = ranks_inc[-1]
    padded = ((counts + BT - 1) // BT) * BT
    pad_start = jnp.concatenate(
        [jnp.zeros((1,), padded.dtype), jnp.cumsum(padded)[:-1]]
    )
    rank = jnp.sum(ranks_inc * onehot, axis=1) - 1
    base = jnp.sum(onehot * pad_start[None, :], axis=1)
    dest = (base + rank).astype(jnp.int32)
    tile_expert = (
        jnp.sum(
            (jnp.arange(ntiles, dtype=jnp.int32)[:, None] * BT
             >= pad_start[None, :].astype(jnp.int32)).astype(jnp.int32),
            axis=1,
        )
        - 1
    )
    tile_expert = jnp.clip(tile_expert, 0, e - 1)
    pos0 = dest[0::k]
    pos1 = dest[1::k]
    firsts = jnp.concatenate(
        [jnp.ones((1,), jnp.int32), (tile_expert[1:] != tile_expert[:-1]).astype(jnp.int32)]
    )
    grp = jnp.cumsum(firsts) - 1
    ngroups = grp[-1] + 1
    gid = jnp.arange(ntiles, dtype=jnp.int32)
    sel = (grp[None, :] == gid[:, None]).astype(jnp.int32) * firsts[None, :]
    ge = jnp.sum(sel * tile_expert[None, :], axis=1)
    nxt_e = jnp.sum(
        ((grp[:, None] + 1) == gid[None, :]).astype(jnp.int32) * ge[None, :], axis=1
    )
    valid = (grp + 1 < ngroups).astype(jnp.int32)
    slot = grp % 2
    sched = jnp.stack(
        [firsts, nxt_e, valid, slot, tile_expert]
    ).astype(jnp.int32)
    return pos0, pos1, sched


def kernel(hidden_states, topk_weights, topk_ids, w1, w3, w2):
    t, d = hidden_states.shape
    e = w1.shape[0]
    k = topk_ids.shape[1]
    n = t * k
    assert n % BT == 0 and k == 2
    ntiles = n // BT + e - 1
    nrows = ntiles * BT

    pos0, pos1, sched = _routing(topk_ids, topk_weights, t, e, k, ntiles, nrows)
    x_sorted = _sc_dispatch_scatter(hidden_states, pos0, pos1, nrows)
    yw = _grouped_mlp(x_sorted, sched, w1, w3, w2)
    w0x = jnp.broadcast_to(topk_weights[:, 0:1], (t, 16))
    w1x = jnp.broadcast_to(topk_weights[:, 1:2], (t, 16))
    return _sc_finalize_combine(yw, pos0, pos1, w0x, w1x, t, d)
    g0, g1 = _sc_finalize_gather(yw, pos0, pos1, t, d)
    return _combine_add(g0, g1, topk_weights[:, 0:1], topk_weights[:, 1:2])
